# baseline (speedup 1.0000x reference)
import functools

import jax
import jax.numpy as jnp
from jax import lax
from jax.experimental import pallas as pl
from jax.experimental.pallas import tpu as pltpu
from jax.experimental.pallas import tpu_sc as plsc

_D = 1024
_F = 512
_E = 64
_T = 2048
_CAP = 80
_S = _E * _CAP
_BUF_ROWS = _S + 8
_NC = 2
_NS = 16
_L = 16
_NW = _NC * _NS
_TPW = _T // _NW
_CHUNK = 32


def _router_body(x_ref, wr_ref, meta_ref, g0_ref, g1_ref):
    x = x_ref[...]
    wr = wr_ref[...]
    logits = jnp.dot(x, wr, preferred_element_type=jnp.float32)
    m = jnp.max(logits, axis=-1, keepdims=True)
    p = jnp.exp(logits - m)
    probs = p / jnp.sum(p, axis=-1, keepdims=True)
    ie = lax.broadcasted_iota(jnp.int32, (_T, _E), 1)
    v0 = jnp.max(probs, axis=-1, keepdims=True)
    i0 = jnp.min(jnp.where(probs == v0, ie, _E), axis=-1, keepdims=True)
    probs2 = jnp.where(ie == i0, -1.0, probs)
    v1 = jnp.max(probs2, axis=-1, keepdims=True)
    i1 = jnp.min(jnp.where(probs2 == v1, ie, _E), axis=-1, keepdims=True)
    denom = v0 + v1
    g0 = v0 / denom
    g1 = v1 / denom
    oh0 = jnp.where(ie == i0, 1.0, 0.0)
    oh1 = jnp.where(ie == i1, 1.0, 0.0)
    a = oh0 + oh1
    nchunk = 16
    rows = _T // nchunk
    ir = lax.broadcasted_iota(jnp.int32, (rows, rows), 0)
    ic = lax.broadcasted_iota(jnp.int32, (rows, rows), 1)
    ltri = jnp.where(ir >= ic, 1.0, 0.0)
    carry = jnp.zeros((1, _E), jnp.float32)
    pieces = []
    for c in range(nchunk):
        ac = lax.slice(a, (c * rows, 0), ((c + 1) * rows, _E))
        pc = jnp.dot(ltri, ac, preferred_element_type=jnp.float32)
        pieces.append(pc - ac + carry)
        carry = carry + lax.slice(pc, (rows - 1, 0), (rows, _E))
    st = jnp.concatenate(pieces, axis=0)
    pos0 = jnp.sum(st * oh0, axis=-1, keepdims=True)
    pos1 = jnp.sum((st + oh0) * oh1, axis=-1, keepdims=True)
    pos0i = pos0.astype(jnp.int32)
    pos1i = pos1.astype(jnp.int32)
    keep0 = pos0i < _CAP
    keep1 = pos1i < _CAP
    d0 = i0 * _CAP + pos0i
    d1 = i1 * _CAP + pos1i
    zero = jnp.zeros_like(d0)
    meta = jnp.concatenate(
        [jnp.where(keep0, d0, _S),
         jnp.where(keep1, d1, _S),
         jnp.where(keep0, d0, 0),
         jnp.where(keep1, d1, 0),
         zero, zero, zero, zero], axis=1)
    meta_ref[...] = jnp.transpose(meta)
    g0_ref[...] = jnp.broadcast_to(jnp.where(keep0, g0, 0.0), (_T, _L))
    g1_ref[...] = jnp.broadcast_to(jnp.where(keep1, g1, 0.0), (_T, _L))


_router = pl.pallas_call(
    _router_body,
    out_shape=[
        jax.ShapeDtypeStruct((8, _T), jnp.int32),
        jax.ShapeDtypeStruct((_T, _L), jnp.float32),
        jax.ShapeDtypeStruct((_T, _L), jnp.float32),
    ],
)


def _wid():
    return lax.axis_index("s") * _NC + lax.axis_index("c")


@functools.cache
def _sc_kernels():
    mesh = plsc.VectorSubcoreMesh(
        core_axis_name="c", subcore_axis_name="s", num_cores=_NC)

    @functools.partial(
        pl.kernel,
        out_type=jax.ShapeDtypeStruct((_BUF_ROWS, _D), jnp.float32),
        mesh=mesh,
        scratch_types=[
            pltpu.VMEM((_TPW,), jnp.int32),
            pltpu.VMEM((_TPW,), jnp.int32),
            pltpu.VMEM((_TPW, _D), jnp.float32),
            pltpu.SemaphoreType.DMA,
            pltpu.SemaphoreType.DMA,
            pltpu.SemaphoreType.DMA,
        ],
    )
    def dispatch(x_hbm, meta_hbm, buf_hbm,
                 idx0_v, idx1_v, rows_v, semx, sem0, sem1):
        base = _wid() * _TPW
        cpx = pltpu.async_copy(x_hbm.at[pl.ds(base, _TPW)], rows_v, semx)
        pltpu.sync_copy(meta_hbm.at[0, pl.ds(base, _TPW)], idx0_v)
        pltpu.sync_copy(meta_hbm.at[1, pl.ds(base, _TPW)], idx1_v)
        cpx.wait()
        cp0 = pltpu.async_copy(rows_v, buf_hbm.at[idx0_v], sem0)
        cp1 = pltpu.async_copy(rows_v, buf_hbm.at[idx1_v], sem1)
        cp0.wait()
        cp1.wait()

    @functools.partial(
        pl.kernel,
        out_type=jax.ShapeDtypeStruct((_T, _D), jnp.float32),
        mesh=mesh,
        scratch_types=[
            pltpu.VMEM((_CHUNK,), jnp.int32),
            pltpu.VMEM((_CHUNK,), jnp.int32),
            pltpu.VMEM((_CHUNK, _L), jnp.float32),
            pltpu.VMEM((_CHUNK, _L), jnp.float32),
            pltpu.VMEM((_CHUNK, _D), jnp.float32),
            pltpu.VMEM((_CHUNK, _D), jnp.float32),
            pltpu.VMEM((_CHUNK, _D), jnp.float32),
            pltpu.SemaphoreType.DMA,
            pltpu.SemaphoreType.DMA,
        ],
    )
    def combine(eout_hbm, meta_hbm, g0_hbm, g1_hbm, out_hbm,
                idx0_v, idx1_v, g0_v, g1_v, r0_v, r1_v, o_v,
                sem0, sem1):
        def chunk_body(ci, _):
            base = _wid() * _TPW + ci * _CHUNK
            pltpu.sync_copy(meta_hbm.at[2, pl.ds(base, _CHUNK)], idx0_v)
            pltpu.sync_copy(meta_hbm.at[3, pl.ds(base, _CHUNK)], idx1_v)
            pltpu.sync_copy(g0_hbm.at[pl.ds(base, _CHUNK)], g0_v)
            pltpu.sync_copy(g1_hbm.at[pl.ds(base, _CHUNK)], g1_v)
            cp0 = pltpu.async_copy(eout_hbm.at[idx0_v], r0_v, sem0)
            cp1 = pltpu.async_copy(eout_hbm.at[idx1_v], r1_v, sem1)
            cp0.wait()
            cp1.wait()

            def tok_body(t, _):
                gv0 = g0_v[t]
                gv1 = g1_v[t]
                for v in range(_D // _L):
                    sl = pl.ds(v * _L, _L)
                    o_v[t, sl] = r0_v[t, sl] * gv0 + r1_v[t, sl] * gv1
                return 0

            lax.fori_loop(0, _CHUNK, tok_body, 0)
            pltpu.sync_copy(o_v, out_hbm.at[pl.ds(base, _CHUNK)])
            return 0

        lax.fori_loop(0, _TPW // _CHUNK, chunk_body, 0)

    return dispatch, combine


_EPB = 4


def _ffn_body(buf_ref, w1_ref, w2_ref, out_ref):
    for i in range(_EPB):
        b = buf_ref[pl.ds(i * _CAP, _CAP), :]
        b = jnp.where(jnp.abs(b) < 1e30, b, 0.0)
        h = jnp.dot(b, w1_ref[i], preferred_element_type=jnp.float32)
        h = h * lax.logistic(h)
        out_ref[pl.ds(i * _CAP, _CAP), :] = jnp.dot(
            h, w2_ref[i], preferred_element_type=jnp.float32)


_ffn = pl.pallas_call(
    _ffn_body,
    grid=(_E // _EPB,),
    in_specs=[
        pl.BlockSpec((_EPB * _CAP, _D), lambda e: (e, 0)),
        pl.BlockSpec((_EPB, _D, _F), lambda e: (e, 0, 0)),
        pl.BlockSpec((_EPB, _F, _D), lambda e: (e, 0, 0)),
    ],
    out_specs=pl.BlockSpec((_EPB * _CAP, _D), lambda e: (e, 0)),
    out_shape=jax.ShapeDtypeStruct((_S, _D), jnp.float32),
)


def kernel(x, w_router, w1, w2):
    dispatch, combine = _sc_kernels()
    meta, g0b, g1b = _router(x, w_router)
    buf = dispatch(x, meta)
    eout = _ffn(buf, w1, w2)
    return combine(eout, meta, g0b, g1b)


# speedup vs baseline: 1.0136x; 1.0136x over previous
import functools

import jax
import jax.numpy as jnp
from jax import lax
from jax.experimental import pallas as pl
from jax.experimental.pallas import tpu as pltpu
from jax.experimental.pallas import tpu_sc as plsc

_D = 1024
_F = 512
_E = 64
_T = 2048
_CAP = 80
_S = _E * _CAP
_EPAD = 2
_BUF_ROWS = (_E + _EPAD) * _CAP
_NC = 2
_NS = 16
_L = 16
_NW = _NC * _NS
_TPW = _T // _NW
_CHUNK = 32
_EPB = 2


def _router_body(x_ref, wr_ref, meta_ref, g_ref):
    x = x_ref[...]
    wr = wr_ref[...]
    logits = jnp.dot(x, wr, preferred_element_type=jnp.float32)
    m = jnp.max(logits, axis=-1, keepdims=True)
    p = jnp.exp(logits - m)
    probs = p / jnp.sum(p, axis=-1, keepdims=True)
    ie = lax.broadcasted_iota(jnp.int32, (_T, _E), 1)
    v0 = jnp.max(probs, axis=-1, keepdims=True)
    i0 = jnp.min(jnp.where(probs == v0, ie, _E), axis=-1, keepdims=True)
    probs2 = jnp.where(ie == i0, -1.0, probs)
    v1 = jnp.max(probs2, axis=-1, keepdims=True)
    i1 = jnp.min(jnp.where(probs2 == v1, ie, _E), axis=-1, keepdims=True)
    denom = v0 + v1
    g0 = v0 / denom
    g1 = v1 / denom
    oh0 = jnp.where(ie == i0, 1.0, 0.0)
    oh1 = jnp.where(ie == i1, 1.0, 0.0)
    a = oh0 + oh1
    nchunk = 16
    rows = _T // nchunk
    ir = lax.broadcasted_iota(jnp.int32, (rows, rows), 0)
    ic = lax.broadcasted_iota(jnp.int32, (rows, rows), 1)
    ltri = jnp.where(ir >= ic, 1.0, 0.0)
    carry = jnp.zeros((1, _E), jnp.float32)
    pieces = []
    for c in range(nchunk):
        ac = lax.slice(a, (c * rows, 0), ((c + 1) * rows, _E))
        pc = jnp.dot(ltri, ac, preferred_element_type=jnp.float32)
        pieces.append(pc - ac + carry)
        carry = carry + lax.slice(pc, (rows - 1, 0), (rows, _E))
    st = jnp.concatenate(pieces, axis=0)
    pos0 = jnp.sum(st * oh0, axis=-1, keepdims=True)
    pos1 = jnp.sum((st + oh0) * oh1, axis=-1, keepdims=True)
    pos0i = pos0.astype(jnp.int32)
    pos1i = pos1.astype(jnp.int32)
    keep0 = pos0i < _CAP
    keep1 = pos1i < _CAP
    d0 = i0 * _CAP + pos0i
    d1 = i1 * _CAP + pos1i
    zero = jnp.zeros_like(d0)
    meta = jnp.concatenate(
        [jnp.where(keep0, d0, _S),
         jnp.where(keep1, d1, _S),
         jnp.where(keep0, d0, _S),
         jnp.where(keep1, d1, _S),
         zero, zero, zero, zero], axis=1)
    meta_ref[...] = jnp.transpose(meta)
    ip = lax.broadcasted_iota(jnp.int32, (_T, _CAP), 1)
    ohp0 = jnp.where(ip == pos0i, 1.0, 0.0)
    ohp1 = jnp.where(ip == pos1i, 1.0, 0.0)
    a0 = oh0 * jnp.where(keep0, g0, 0.0)
    a1 = oh1 * jnp.where(keep1, g1, 0.0)
    dn = (((0,), (0,)), ((), ()))
    gm = (lax.dot_general(a0, ohp0, dn, precision=lax.Precision.HIGHEST)
          + lax.dot_general(a1, ohp1, dn, precision=lax.Precision.HIGHEST))
    gfull = jnp.concatenate(
        [gm, jnp.zeros((_EPAD, _CAP), jnp.float32)], axis=0)
    g_ref[...] = gfull.reshape((_E + _EPAD) // _EPB, _EPB, _CAP)


_router = pl.pallas_call(
    _router_body,
    out_shape=[
        jax.ShapeDtypeStruct((8, _T), jnp.int32),
        jax.ShapeDtypeStruct(((_E + _EPAD) // _EPB, _EPB, _CAP), jnp.float32),
    ],
)


def _wid():
    return lax.axis_index("s") * _NC + lax.axis_index("c")


@functools.cache
def _sc_kernels():
    mesh = plsc.VectorSubcoreMesh(
        core_axis_name="c", subcore_axis_name="s", num_cores=_NC)

    @functools.partial(
        pl.kernel,
        out_type=jax.ShapeDtypeStruct((_BUF_ROWS, _D), jnp.float32),
        mesh=mesh,
        scratch_types=[
            pltpu.VMEM((_TPW,), jnp.int32),
            pltpu.VMEM((_TPW,), jnp.int32),
            pltpu.VMEM((_TPW, _D), jnp.float32),
            pltpu.SemaphoreType.DMA,
            pltpu.SemaphoreType.DMA,
            pltpu.SemaphoreType.DMA,
        ],
    )
    def dispatch(x_hbm, meta_hbm, buf_hbm,
                 idx0_v, idx1_v, rows_v, semx, sem0, sem1):
        base = _wid() * _TPW
        cpx = pltpu.async_copy(x_hbm.at[pl.ds(base, _TPW)], rows_v, semx)
        pltpu.sync_copy(meta_hbm.at[0, pl.ds(base, _TPW)], idx0_v)
        pltpu.sync_copy(meta_hbm.at[1, pl.ds(base, _TPW)], idx1_v)
        cpx.wait()
        cp0 = pltpu.async_copy(rows_v, buf_hbm.at[idx0_v], sem0)
        cp1 = pltpu.async_copy(rows_v, buf_hbm.at[idx1_v], sem1)
        cp0.wait()
        cp1.wait()

    @functools.partial(
        pl.kernel,
        out_type=jax.ShapeDtypeStruct((_T, _D), jnp.float32),
        mesh=mesh,
        scratch_types=[
            pltpu.VMEM((_CHUNK,), jnp.int32),
            pltpu.VMEM((_CHUNK,), jnp.int32),
            pltpu.VMEM((_CHUNK, _D), jnp.float32),
            pltpu.VMEM((_CHUNK, _D), jnp.float32),
            pltpu.VMEM((_CHUNK, _D), jnp.float32),
            pltpu.SemaphoreType.DMA,
            pltpu.SemaphoreType.DMA,
            pltpu.SemaphoreType.DMA,
        ],
    )
    def combine(eout_hbm, meta_hbm, out_hbm,
                idx0_v, idx1_v, r0_v, r1_v, o_v,
                sem0, sem1, semo):
        def chunk_body(ci, _):
            base = _wid() * _TPW + ci * _CHUNK
            pltpu.sync_copy(meta_hbm.at[2, pl.ds(base, _CHUNK)], idx0_v)
            pltpu.sync_copy(meta_hbm.at[3, pl.ds(base, _CHUNK)], idx1_v)
            cp0 = pltpu.async_copy(eout_hbm.at[idx0_v], r0_v, sem0)
            cp1 = pltpu.async_copy(eout_hbm.at[idx1_v], r1_v, sem1)
            cp0.wait()
            cp1.wait()

            def tok_body(t, _):
                for v in range(_D // _L):
                    sl = pl.ds(v * _L, _L)
                    o_v[t, sl] = r0_v[t, sl] + r1_v[t, sl]
                return 0

            lax.fori_loop(0, _CHUNK, tok_body, 0)
            pltpu.sync_copy(o_v, out_hbm.at[pl.ds(base, _CHUNK)])
            return 0

        lax.fori_loop(0, _TPW // _CHUNK, chunk_body, 0)

    return dispatch, combine


def _ffn_body(buf_ref, w1_ref, w2_ref, g_ref, out_ref):
    for i in range(_EPB):
        b = buf_ref[pl.ds(i * _CAP, _CAP), :]
        b = jnp.where(jnp.abs(b) < 1e30, b, 0.0)
        h = jnp.dot(b, w1_ref[i], preferred_element_type=jnp.float32)
        h = h * lax.logistic(h)
        o = jnp.dot(h, w2_ref[i], preferred_element_type=jnp.float32)
        out_ref[pl.ds(i * _CAP, _CAP), :] = o * g_ref[0, i][:, None]


_ffn = pl.pallas_call(
    _ffn_body,
    grid=((_E + _EPAD) // _EPB,),
    in_specs=[
        pl.BlockSpec((_EPB * _CAP, _D), lambda e: (e, 0)),
        pl.BlockSpec((_EPB, _D, _F),
                     lambda e: (jnp.minimum(e, _E // _EPB - 1), 0, 0)),
        pl.BlockSpec((_EPB, _F, _D),
                     lambda e: (jnp.minimum(e, _E // _EPB - 1), 0, 0)),
        pl.BlockSpec((1, _EPB, _CAP), lambda e: (e, 0, 0)),
    ],
    out_specs=pl.BlockSpec((_EPB * _CAP, _D), lambda e: (e, 0)),
    out_shape=jax.ShapeDtypeStruct((_BUF_ROWS, _D), jnp.float32),
)


def kernel(x, w_router, w1, w2):
    dispatch, combine = _sc_kernels()
    meta, g = _router(x, w_router)
    buf = dispatch(x, meta)
    eout = _ffn(buf, w1, w2, g)
    return combine(eout, meta)
